# speedup vs baseline: 3.8807x; 3.8807x over previous
import functools

import jax
import jax.numpy as jnp
from jax import lax
from jax.experimental import pallas as pl
from jax.experimental.pallas import tpu as pltpu
from jax.experimental.pallas import tpu_sc as plsc

_NUM_VQ = 4
_VOCAB = 626
_D = 768


def _sum_tables_body(emb_ref, out_ref):
    out_ref[...] = (emb_ref[0] + emb_ref[1]) + (emb_ref[2] + emb_ref[3])


def _sum_tables(emb_code):
    return pl.pallas_call(
        _sum_tables_body,
        out_shape=jax.ShapeDtypeStruct((_VOCAB, _D), jnp.float32),
    )(emb_code)


@functools.lru_cache(maxsize=None)
def _make_gather(B):
    info = plsc.get_sparse_core_info()
    NC, NS = info.num_cores, info.num_subcores
    NW = NC * NS
    assert B % (8 * NW) == 0
    b_per_w = B // NW
    C = 64
    n_chunks = b_per_w // C
    mesh = plsc.VectorSubcoreMesh(core_axis_name="c", subcore_axis_name="s")

    @functools.partial(
        pl.kernel,
        mesh=mesh,
        out_type=jax.ShapeDtypeStruct((B, _D), jnp.float32),
        scratch_types=[
            pltpu.VMEM((b_per_w,), jnp.int32),
            pltpu.VMEM((2, C, _D), jnp.float32),
            pltpu.SemaphoreType.DMA,
            pltpu.SemaphoreType.DMA,
            pltpu.SemaphoreType.DMA,
        ],
    )
    def gather_kernel(table_hbm, idx_hbm, out_hbm, idx_v, rows_v, sem_g,
                      sem_s0, sem_s1):
        wid = lax.axis_index("s") * NC + lax.axis_index("c")
        base = wid * b_per_w
        pltpu.sync_copy(idx_hbm.at[pl.ds(base, b_per_w)], idx_v)
        store_sems = (sem_s0, sem_s1)
        gathers = [None] * n_chunks
        stores = [None] * n_chunks
        gathers[0] = pltpu.async_copy(
            table_hbm.at[idx_v.at[pl.ds(0, C)]], rows_v.at[0], sem_g)
        for g in range(n_chunks):
            b = g & 1
            gathers[g].wait()
            stores[g] = pltpu.async_copy(
                rows_v.at[b], out_hbm.at[pl.ds(base + g * C, C)],
                store_sems[b])
            if g + 1 < n_chunks:
                if g >= 1:
                    stores[g - 1].wait()
                gathers[g + 1] = pltpu.async_copy(
                    table_hbm.at[idx_v.at[pl.ds((g + 1) * C, C)]],
                    rows_v.at[1 - b], sem_g)
        stores[n_chunks - 1].wait()

    return gather_kernel


def kernel(input_ids, emb_code):
    B, T = input_ids.shape
    table = _sum_tables(emb_code)
    idx = input_ids.reshape(-1).astype(jnp.int32)
    out = _make_gather(B * T)(table, idx)
    return out.reshape(B, T, _D)


# speedup vs baseline: 3.9312x; 1.0130x over previous
import functools

import jax
import jax.numpy as jnp
from jax import lax
from jax.experimental import pallas as pl
from jax.experimental.pallas import tpu as pltpu
from jax.experimental.pallas import tpu_sc as plsc

_NUM_VQ = 4
_VOCAB = 626
_D = 768


def _sum_tables_body(emb_ref, out_ref):
    out_ref[...] = (emb_ref[0] + emb_ref[1]) + (emb_ref[2] + emb_ref[3])


def _sum_tables(emb_code):
    return pl.pallas_call(
        _sum_tables_body,
        out_shape=jax.ShapeDtypeStruct((_VOCAB, _D), jnp.float32),
    )(emb_code)


@functools.lru_cache(maxsize=None)
def _make_gather(B):
    info = plsc.get_sparse_core_info()
    NC, NS = info.num_cores, info.num_subcores
    NW = NC * NS
    assert B % (8 * NW) == 0
    b_per_w = B // NW
    C = 32
    NBUF = 4
    n_chunks = b_per_w // C
    mesh = plsc.VectorSubcoreMesh(core_axis_name="c", subcore_axis_name="s")

    @functools.partial(
        pl.kernel,
        mesh=mesh,
        out_type=jax.ShapeDtypeStruct((B, _D), jnp.float32),
        scratch_types=[
            pltpu.VMEM((b_per_w,), jnp.int32),
            pltpu.VMEM((NBUF, C, _D), jnp.float32),
        ] + [pltpu.SemaphoreType.DMA] * (2 * NBUF),
    )
    def gather_kernel(table_hbm, idx_hbm, out_hbm, idx_v, rows_v, *sems):
        sem_g, sem_s = sems[:NBUF], sems[NBUF:]
        wid = lax.axis_index("s") * NC + lax.axis_index("c")
        base = wid * b_per_w
        pltpu.sync_copy(idx_hbm.at[pl.ds(base, b_per_w)], idx_v)

        def start_gather(g):
            return pltpu.async_copy(
                table_hbm.at[idx_v.at[pl.ds(g * C, C)]],
                rows_v.at[g % NBUF], sem_g[g % NBUF])

        gathers = [None] * n_chunks
        stores = [None] * n_chunks
        for g in range(NBUF - 1):
            gathers[g] = start_gather(g)
        for g in range(n_chunks):
            b = g % NBUF
            gathers[g].wait()
            stores[g] = pltpu.async_copy(
                rows_v.at[b], out_hbm.at[pl.ds(base + g * C, C)], sem_s[b])
            nxt = g + NBUF - 1
            if nxt < n_chunks:
                if g >= 1:
                    stores[g - 1].wait()
                gathers[nxt] = start_gather(nxt)
        for g in range(max(0, n_chunks - NBUF), n_chunks):
            stores[g].wait()

    return gather_kernel


def kernel(input_ids, emb_code):
    B, T = input_ids.shape
    table = _sum_tables(emb_code)
    idx = input_ids.reshape(-1).astype(jnp.int32)
    out = _make_gather(B * T)(table, idx)
    return out.reshape(B, T, _D)


# speedup vs baseline: 4.5880x; 1.1671x over previous
import functools

import jax
import jax.numpy as jnp
from jax import lax
from jax.experimental import pallas as pl
from jax.experimental.pallas import tpu as pltpu
from jax.experimental.pallas import tpu_sc as plsc

_NUM_VQ = 4
_VOCAB = 626
_D = 768


def _sum_tables_body(emb_ref, out_ref):
    out_ref[...] = ((emb_ref[:, 0, :] + emb_ref[:, 1, :])
                    + (emb_ref[:, 2, :] + emb_ref[:, 3, :]))


def _sum_tables(emb_code):
    tt = jnp.transpose(emb_code, (1, 0, 2))
    return pl.pallas_call(
        _sum_tables_body,
        out_shape=jax.ShapeDtypeStruct((_VOCAB, _D), jnp.float32),
    )(tt)


@functools.lru_cache(maxsize=None)
def _make_gather(B):
    info = plsc.get_sparse_core_info()
    NC, NS = info.num_cores, info.num_subcores
    NW = NC * NS
    assert B % (8 * NW) == 0
    b_per_w = B // NW
    C = 32
    NBUF = 4
    n_chunks = b_per_w // C
    mesh = plsc.VectorSubcoreMesh(core_axis_name="c", subcore_axis_name="s")

    @functools.partial(
        pl.kernel,
        mesh=mesh,
        out_type=jax.ShapeDtypeStruct((B, _D), jnp.float32),
        scratch_types=[
            pltpu.VMEM((b_per_w,), jnp.int32),
            pltpu.VMEM((NBUF, C, _D), jnp.float32),
        ] + [pltpu.SemaphoreType.DMA] * (2 * NBUF),
    )
    def gather_kernel(table_hbm, idx_hbm, out_hbm, idx_v, rows_v, *sems):
        sem_g, sem_s = sems[:NBUF], sems[NBUF:]
        wid = lax.axis_index("s") * NC + lax.axis_index("c")
        base = wid * b_per_w
        pltpu.sync_copy(idx_hbm.at[pl.ds(base, b_per_w)], idx_v)

        def start_gather(g):
            return pltpu.async_copy(
                table_hbm.at[idx_v.at[pl.ds(g * C, C)]],
                rows_v.at[g % NBUF], sem_g[g % NBUF])

        gathers = [None] * n_chunks
        stores = [None] * n_chunks
        for g in range(NBUF - 1):
            gathers[g] = start_gather(g)
        for g in range(n_chunks):
            b = g % NBUF
            gathers[g].wait()
            stores[g] = pltpu.async_copy(
                rows_v.at[b], out_hbm.at[pl.ds(base + g * C, C)], sem_s[b])
            nxt = g + NBUF - 1
            if nxt < n_chunks:
                if g >= 1:
                    stores[g - 1].wait()
                gathers[nxt] = start_gather(nxt)
        for g in range(max(0, n_chunks - NBUF), n_chunks):
            stores[g].wait()

    return gather_kernel


def kernel(input_ids, emb_code):
    B, T = input_ids.shape
    table = _sum_tables(emb_code)
    idx = input_ids.reshape(-1).astype(jnp.int32)
    out = _make_gather(B * T)(table, idx)
    return out.reshape(B, T, _D)


# speedup vs baseline: 4.6890x; 1.0220x over previous
import functools

import jax
import jax.numpy as jnp
from jax import lax
from jax.experimental import pallas as pl
from jax.experimental.pallas import tpu as pltpu
from jax.experimental.pallas import tpu_sc as plsc

_NUM_VQ = 4
_VOCAB = 626
_D = 768


def _sum_tables_body(emb_ref, out_ref):
    out_ref[...] = ((emb_ref[:, 0, :] + emb_ref[:, 1, :])
                    + (emb_ref[:, 2, :] + emb_ref[:, 3, :]))


def _sum_tables(emb_code):
    tt = jnp.transpose(emb_code, (1, 0, 2))
    return pl.pallas_call(
        _sum_tables_body,
        out_shape=jax.ShapeDtypeStruct((_VOCAB, _D), jnp.float32),
    )(tt)


@functools.lru_cache(maxsize=None)
def _make_gather(B):
    info = plsc.get_sparse_core_info()
    NC, NS = info.num_cores, info.num_subcores
    NW = NC * NS
    assert B % (8 * NW) == 0
    b_per_w = B // NW
    C = 32
    NBUF = 5
    n_chunks = b_per_w // C
    mesh = plsc.VectorSubcoreMesh(core_axis_name="c", subcore_axis_name="s")

    @functools.partial(
        pl.kernel,
        mesh=mesh,
        out_type=jax.ShapeDtypeStruct((B, _D), jnp.float32),
        scratch_types=[
            pltpu.VMEM((b_per_w,), jnp.int32),
            pltpu.VMEM((NBUF, C, _D), jnp.float32),
        ] + [pltpu.SemaphoreType.DMA] * (2 * NBUF),
    )
    def gather_kernel(table_hbm, idx_hbm, out_hbm, idx_v, rows_v, *sems):
        sem_g, sem_s = sems[:NBUF], sems[NBUF:]
        wid = lax.axis_index("s") * NC + lax.axis_index("c")
        base = wid * b_per_w
        pltpu.sync_copy(idx_hbm.at[pl.ds(base, b_per_w)], idx_v)

        def start_gather(g):
            return pltpu.async_copy(
                table_hbm.at[idx_v.at[pl.ds(g * C, C)]],
                rows_v.at[g % NBUF], sem_g[g % NBUF])

        gathers = [None] * n_chunks
        stores = [None] * n_chunks
        for g in range(NBUF - 1):
            gathers[g] = start_gather(g)
        for g in range(n_chunks):
            b = g % NBUF
            gathers[g].wait()
            stores[g] = pltpu.async_copy(
                rows_v.at[b], out_hbm.at[pl.ds(base + g * C, C)], sem_s[b])
            nxt = g + NBUF - 1
            if nxt < n_chunks:
                if g >= 1:
                    stores[g - 1].wait()
                gathers[nxt] = start_gather(nxt)
        for g in range(max(0, n_chunks - NBUF), n_chunks):
            stores[g].wait()

    return gather_kernel


def kernel(input_ids, emb_code):
    B, T = input_ids.shape
    table = _sum_tables(emb_code)
    idx = input_ids.reshape(-1).astype(jnp.int32)
    out = _make_gather(B * T)(table, idx)
    return out.reshape(B, T, _D)


# speedup vs baseline: 4.8137x; 1.0266x over previous
import functools

import jax
import jax.numpy as jnp
from jax import lax
from jax.experimental import pallas as pl
from jax.experimental.pallas import tpu as pltpu
from jax.experimental.pallas import tpu_sc as plsc

_NUM_VQ = 4
_VOCAB = 626
_D = 768


def _sum_tables_body(emb_ref, out_ref):
    out_ref[...] = ((emb_ref[:, 0, :] + emb_ref[:, 1, :])
                    + (emb_ref[:, 2, :] + emb_ref[:, 3, :]))


def _sum_tables(emb_code):
    tt = jnp.transpose(emb_code, (1, 0, 2))
    return pl.pallas_call(
        _sum_tables_body,
        out_shape=jax.ShapeDtypeStruct((_VOCAB, _D), jnp.float32),
    )(tt)


@functools.lru_cache(maxsize=None)
def _make_gather(BB, T):
    info = plsc.get_sparse_core_info()
    NC, NS = info.num_cores, info.num_subcores
    NW = NC * NS
    B = BB * T
    assert B % (8 * NW) == 0
    b_per_w = B // NW
    w_per_row = T // b_per_w
    C = 32
    NBUF = 5
    n_chunks = b_per_w // C
    mesh = plsc.VectorSubcoreMesh(core_axis_name="c", subcore_axis_name="s")

    @functools.partial(
        pl.kernel,
        mesh=mesh,
        out_type=jax.ShapeDtypeStruct((B, _D), jnp.float32),
        scratch_types=[
            pltpu.VMEM((b_per_w,), jnp.int32),
            pltpu.VMEM((NBUF, C, _D), jnp.float32),
        ] + [pltpu.SemaphoreType.DMA] * (2 * NBUF),
    )
    def gather_kernel(table_hbm, idx_hbm, out_hbm, idx_v, rows_v, *sems):
        sem_g, sem_s = sems[:NBUF], sems[NBUF:]
        wid = lax.axis_index("s") * NC + lax.axis_index("c")
        base = wid * b_per_w
        pltpu.sync_copy(
            idx_hbm.at[wid // w_per_row,
                       pl.ds((wid % w_per_row) * b_per_w, b_per_w)], idx_v)

        def start_gather(g):
            return pltpu.async_copy(
                table_hbm.at[idx_v.at[pl.ds(g * C, C)]],
                rows_v.at[g % NBUF], sem_g[g % NBUF])

        gathers = [None] * n_chunks
        stores = [None] * n_chunks
        for g in range(NBUF - 1):
            gathers[g] = start_gather(g)
        for g in range(n_chunks):
            b = g % NBUF
            gathers[g].wait()
            stores[g] = pltpu.async_copy(
                rows_v.at[b], out_hbm.at[pl.ds(base + g * C, C)], sem_s[b])
            nxt = g + NBUF - 1
            if nxt < n_chunks:
                if g >= 1:
                    stores[g - 1].wait()
                gathers[nxt] = start_gather(nxt)
        for g in range(max(0, n_chunks - NBUF), n_chunks):
            stores[g].wait()

    return gather_kernel


def kernel(input_ids, emb_code):
    B, T = input_ids.shape
    table = _sum_tables(emb_code)
    out = _make_gather(B, T)(table, input_ids.astype(jnp.int32))
    return out.reshape(B, T, _D)
